# baseline (speedup 1.0000x reference)
import functools

import jax
import jax.numpy as jnp
from jax import lax
from jax.experimental import pallas as pl
from jax.experimental.pallas import tpu as pltpu
from jax.experimental.pallas import tpu_sc as plsc

_K = 20
_KPAD = 32


def _knn_tc_kernel(xt_ref, w2t_ref, wdt_ref, b_ref, idx_ref, y_ref, c_ref,
                   *, tile, k):
    n = xt_ref.shape[1]
    i = pl.program_id(1)
    bb = pl.program_id(0)

    xt = xt_ref[0]
    xt_tile = xt_ref[0, pl.ds(i * tile, tile), :]

    sq = jnp.sum(xt * xt, axis=1)
    sq_tile = jnp.sum(xt_tile * xt_tile, axis=1)
    xi = -2.0 * lax.dot_general(
        xt_tile, xt, (((1,), (1,)), ((), ())),
        preferred_element_type=jnp.float32)
    dist = (xi + sq_tile[:, None]) + sq[None, :]

    colf = lax.broadcasted_iota(jnp.int32, (tile, n), 1).astype(jnp.float32)
    col32 = lax.broadcasted_iota(jnp.int32, (tile, _KPAD), 1)

    def extract_min(d):
        m = jnp.min(d, axis=1, keepdims=True)
        idxf = jnp.min(jnp.where(d == m, colf, float(n)), axis=1,
                       keepdims=True)
        return idxf, jnp.where(colf == idxf, jnp.inf, d)

    _, dist = extract_min(dist)

    def body(j, carry):
        d, acc = carry
        idxf, d = extract_min(d)
        idx = idxf.astype(jnp.int32)
        acc = jnp.where(col32 == j, idx + bb * n, acc)
        return d, acc

    acc0 = jnp.zeros((tile, _KPAD), dtype=jnp.int32)
    _, idxacc = lax.fori_loop(0, k, body, (dist, acc0))
    idx_ref[0] = idxacc

    y_ref[0] = lax.dot_general(
        xt_tile, w2t_ref[...], (((1,), (0,)), ((), ())),
        preferred_element_type=jnp.float32)
    c = lax.dot_general(
        xt_tile, wdt_ref[...], (((1,), (0,)), ((), ())),
        preferred_element_type=jnp.float32)
    c_ref[0] = c + b_ref[...]


def _make_sc_gather_max(bn, n, fout, k, pts_w, chunk):
    nsteps = pts_w // chunk
    mesh = plsc.VectorSubcoreMesh(core_axis_name="c", subcore_axis_name="s")

    @functools.partial(
        pl.kernel, mesh=mesh,
        out_type=jax.ShapeDtypeStruct((bn, fout), jnp.float32),
        compiler_params=pltpu.CompilerParams(use_tc_tiling_on_sc=False),
        scratch_types=[
            pltpu.VMEM((k, chunk), jnp.int32),
            pltpu.VMEM((k * chunk, fout), jnp.float32),
            pltpu.VMEM((chunk, fout), jnp.float32),
            pltpu.VMEM((chunk, fout), jnp.float32),
            pltpu.SemaphoreType.DMA,
        ],
    )
    def sc_fn(y_hbm, idxj_hbm, c_hbm, out_hbm, idx_v, rows_v, c_v, out_v, sem):
        wid = lax.axis_index("s") * 2 + lax.axis_index("c")
        base = wid * pts_w
        b = base // n

        def chunk_body(g, carry):
            pt = base + g * chunk
            n0 = pt - b * n
            pltpu.sync_copy(
                idxj_hbm.at[pl.ds(b * _KPAD, k), pl.ds(n0, chunk)], idx_v)
            copies = [
                pltpu.async_copy(
                    y_hbm.at[idx_v.at[j]],
                    rows_v.at[pl.ds(j * chunk, chunk)], sem)
                for j in range(k)
            ]
            pltpu.sync_copy(c_hbm.at[pl.ds(pt, chunk)], c_v)
            for cp in copies:
                cp.wait()

            def point_body(p, carry2):
                for l in range(fout // 16):
                    sl = pl.ds(l * 16, 16)
                    acc = rows_v[p, sl]
                    for j in range(1, k):
                        acc = jnp.maximum(acc, rows_v[j * chunk + p, sl])
                    out_v[p, sl] = acc + c_v[p, sl]
                return carry2

            lax.fori_loop(0, chunk, point_body, 0)
            pltpu.sync_copy(out_v, out_hbm.at[pl.ds(pt, chunk)])
            return carry

        lax.fori_loop(0, nsteps, chunk_body, 0)

    return sc_fn


def kernel(x, W, b):
    B, Fin, N = x.shape
    Fout = W.shape[0]
    tile = 1024 if N % 1024 == 0 else N

    xt = jnp.transpose(x, (0, 2, 1))
    W1, W2 = W[:, :Fin], W[:, Fin:]
    w2t = jnp.transpose(W2)
    wdt = jnp.transpose(W1 - W2)
    b2 = b[None, :]

    idx, y, c = pl.pallas_call(
        functools.partial(_knn_tc_kernel, tile=tile, k=_K),
        grid=(B, N // tile),
        in_specs=[
            pl.BlockSpec((1, N, Fin), lambda bb, ii: (bb, 0, 0)),
            pl.BlockSpec((Fin, Fout), lambda bb, ii: (0, 0)),
            pl.BlockSpec((Fin, Fout), lambda bb, ii: (0, 0)),
            pl.BlockSpec((1, Fout), lambda bb, ii: (0, 0)),
        ],
        out_specs=[
            pl.BlockSpec((1, tile, _KPAD), lambda bb, ii: (bb, ii, 0)),
            pl.BlockSpec((1, tile, Fout), lambda bb, ii: (bb, ii, 0)),
            pl.BlockSpec((1, tile, Fout), lambda bb, ii: (bb, ii, 0)),
        ],
        out_shape=[
            jax.ShapeDtypeStruct((B, N, _KPAD), jnp.int32),
            jax.ShapeDtypeStruct((B, N, Fout), jnp.float32),
            jax.ShapeDtypeStruct((B, N, Fout), jnp.float32),
        ],
    )(xt, w2t, wdt, b2)

    bn = B * N
    idxj = jnp.transpose(idx, (0, 2, 1)).reshape(B * _KPAD, N)
    y_flat = y.reshape(bn, Fout)
    c_flat = c.reshape(bn, Fout)

    nw = 32
    pts_w = bn // nw
    chunk = 32
    sc_fn = _make_sc_gather_max(bn, N, Fout, _K, pts_w, chunk)
    out = sc_fn(y_flat, idxj, c_flat)

    return jnp.transpose(out.reshape(B, N, Fout), (0, 2, 1))


# speedup vs baseline: 1.0746x; 1.0746x over previous
import functools

import jax
import jax.numpy as jnp
from jax import lax
from jax.experimental import pallas as pl
from jax.experimental.pallas import tpu as pltpu
from jax.experimental.pallas import tpu_sc as plsc

_K = 20
_KPAD = 32


def _knn_tc_kernel(xt_ref, w2t_ref, wdt_ref, b_ref, idx_ref, y_ref, c_ref,
                   *, tile, k):
    n = xt_ref.shape[1]
    i = pl.program_id(1)
    bb = pl.program_id(0)

    xt = xt_ref[0]
    xt_tile = xt_ref[0, pl.ds(i * tile, tile), :]

    sq = jnp.sum(xt * xt, axis=1)
    sq_tile = jnp.sum(xt_tile * xt_tile, axis=1)
    xi = -2.0 * lax.dot_general(
        xt_tile, xt, (((1,), (1,)), ((), ())),
        preferred_element_type=jnp.float32)
    dist = (xi + sq_tile[:, None]) + sq[None, :]

    colf = lax.broadcasted_iota(jnp.int32, (tile, n), 1).astype(jnp.float32)
    col32 = lax.broadcasted_iota(jnp.int32, (tile, _KPAD), 1)

    def extract_min(d):
        m = jnp.min(d, axis=1, keepdims=True)
        idxf = jnp.min(jnp.where(d == m, colf, float(n)), axis=1,
                       keepdims=True)
        return idxf, jnp.where(colf == idxf, jnp.inf, d)

    _, dist = extract_min(dist)

    def body(j, carry):
        d, acc = carry
        idxf, d = extract_min(d)
        idx = idxf.astype(jnp.int32)
        acc = jnp.where(col32 == j, idx + bb * n, acc)
        return d, acc

    acc0 = jnp.zeros((tile, _KPAD), dtype=jnp.int32)
    _, idxacc = lax.fori_loop(0, k, body, (dist, acc0))
    idx_ref[0] = idxacc

    y_ref[0] = lax.dot_general(
        xt_tile, w2t_ref[...], (((1,), (0,)), ((), ())),
        preferred_element_type=jnp.float32)
    c = lax.dot_general(
        xt_tile, wdt_ref[...], (((1,), (0,)), ((), ())),
        preferred_element_type=jnp.float32)
    c_ref[0] = c + b_ref[...]


def _make_sc_gather_max(bn, n, fout, k, pts_w, chunk):
    nsteps = pts_w // chunk
    mesh = plsc.VectorSubcoreMesh(core_axis_name="c", subcore_axis_name="s")

    @functools.partial(
        pl.kernel, mesh=mesh,
        out_type=jax.ShapeDtypeStruct((bn, fout), jnp.float32),
        compiler_params=pltpu.CompilerParams(use_tc_tiling_on_sc=False),
        scratch_types=[
            pltpu.VMEM((k, chunk), jnp.int32),
            pltpu.VMEM((k * chunk, fout), jnp.float32),
            pltpu.VMEM((chunk, fout), jnp.float32),
            pltpu.VMEM((chunk, fout), jnp.float32),
            pltpu.SemaphoreType.DMA,
        ],
    )
    def sc_fn(y_hbm, idxj_hbm, c_hbm, out_hbm, idx_v, rows_v, c_v, out_v, sem):
        wid = lax.axis_index("s") * 2 + lax.axis_index("c")
        base = wid * pts_w
        b = base // n

        def chunk_body(g, carry):
            pt = base + g * chunk
            n0 = pt - b * n
            pltpu.sync_copy(
                idxj_hbm.at[pl.ds(b * _KPAD, k), pl.ds(n0, chunk)], idx_v)
            copies = [
                pltpu.async_copy(
                    y_hbm.at[idx_v.at[j]],
                    rows_v.at[pl.ds(j * chunk, chunk)], sem)
                for j in range(k)
            ]
            pltpu.sync_copy(c_hbm.at[pl.ds(pt, chunk)], c_v)
            for cp in copies:
                cp.wait()

            def point_body(p, carry2):
                for l in range(fout // 16):
                    sl = pl.ds(l * 16, 16)
                    acc = rows_v[p, sl]
                    for j in range(1, k):
                        acc = jnp.maximum(acc, rows_v[j * chunk + p, sl])
                    out_v[p, sl] = acc + c_v[p, sl]
                return carry2

            lax.fori_loop(0, chunk, point_body, 0)
            pltpu.sync_copy(out_v, out_hbm.at[pl.ds(pt, chunk)])
            return carry

        lax.fori_loop(0, nsteps, chunk_body, 0)

    return sc_fn


def _half(xt, w2t, wdt, b2, sc_fn, tile):
    B, N, Fin = xt.shape
    Fout = w2t.shape[1]
    idx, y, c = pl.pallas_call(
        functools.partial(_knn_tc_kernel, tile=tile, k=_K),
        grid=(B, N // tile),
        in_specs=[
            pl.BlockSpec((1, N, Fin), lambda bb, ii: (bb, 0, 0)),
            pl.BlockSpec((Fin, Fout), lambda bb, ii: (0, 0)),
            pl.BlockSpec((Fin, Fout), lambda bb, ii: (0, 0)),
            pl.BlockSpec((1, Fout), lambda bb, ii: (0, 0)),
        ],
        out_specs=[
            pl.BlockSpec((1, tile, _KPAD), lambda bb, ii: (bb, ii, 0)),
            pl.BlockSpec((1, tile, Fout), lambda bb, ii: (bb, ii, 0)),
            pl.BlockSpec((1, tile, Fout), lambda bb, ii: (bb, ii, 0)),
        ],
        out_shape=[
            jax.ShapeDtypeStruct((B, N, _KPAD), jnp.int32),
            jax.ShapeDtypeStruct((B, N, Fout), jnp.float32),
            jax.ShapeDtypeStruct((B, N, Fout), jnp.float32),
        ],
    )(xt, w2t, wdt, b2)

    bn = B * N
    idxj = jnp.transpose(idx, (0, 2, 1)).reshape(B * _KPAD, N)
    return sc_fn(y.reshape(bn, Fout), idxj, c.reshape(bn, Fout))


def kernel(x, W, b):
    B, Fin, N = x.shape
    Fout = W.shape[0]
    tile = 512 if N % 512 == 0 else N

    xt = jnp.transpose(x, (0, 2, 1))
    W1, W2 = W[:, :Fin], W[:, Fin:]
    w2t = jnp.transpose(W2)
    wdt = jnp.transpose(W1 - W2)
    b2 = b[None, :]

    bh = B // 2 if B % 2 == 0 else B
    nh = B // bh
    bn = bh * N
    sc_fn = _make_sc_gather_max(bn, N, Fout, _K, bn // 32, 32)
    outs = [_half(xt[h * bh:(h + 1) * bh], w2t, wdt, b2, sc_fn, tile)
            for h in range(nh)]
    out = jnp.concatenate(outs, axis=0)

    return jnp.transpose(out.reshape(B, N, Fout), (0, 2, 1))


# speedup vs baseline: 1.0909x; 1.0152x over previous
import functools

import jax
import jax.numpy as jnp
from jax import lax
from jax.experimental import pallas as pl
from jax.experimental.pallas import tpu as pltpu
from jax.experimental.pallas import tpu_sc as plsc

_K = 20
_KPAD = 32


def _knn_tc_kernel(xt_ref, w2t_ref, wdt_ref, b_ref, idx_ref, y_ref, c_ref,
                   *, tile, k):
    n = xt_ref.shape[1]
    i = pl.program_id(1)
    bb = pl.program_id(0)

    xt = xt_ref[0]
    xt_tile = xt_ref[0, pl.ds(i * tile, tile), :]

    sq = jnp.sum(xt * xt, axis=1)
    sq_tile = jnp.sum(xt_tile * xt_tile, axis=1)
    xi = -2.0 * lax.dot_general(
        xt_tile, xt, (((1,), (1,)), ((), ())),
        preferred_element_type=jnp.float32)
    dist = (xi + sq_tile[:, None]) + sq[None, :]

    colf = lax.broadcasted_iota(jnp.int32, (tile, n), 1).astype(jnp.float32)
    col32 = lax.broadcasted_iota(jnp.int32, (tile, _KPAD), 1)

    def extract_min(d):
        m = jnp.min(d, axis=1, keepdims=True)
        idxf = jnp.min(jnp.where(d == m, colf, float(n)), axis=1,
                       keepdims=True)
        return idxf, jnp.where(colf == idxf, jnp.inf, d)

    _, dist = extract_min(dist)

    def body(j, carry):
        d, acc = carry
        idxf, d = extract_min(d)
        idx = idxf.astype(jnp.int32)
        acc = jnp.where(col32 == j, idx + bb * n, acc)
        return d, acc

    acc0 = jnp.zeros((tile, _KPAD), dtype=jnp.int32)
    _, idxacc = lax.fori_loop(0, k, body, (dist, acc0))
    idx_ref[0] = idxacc

    y_ref[0] = lax.dot_general(
        xt_tile, w2t_ref[...], (((1,), (0,)), ((), ())),
        preferred_element_type=jnp.float32)
    c = lax.dot_general(
        xt_tile, wdt_ref[...], (((1,), (0,)), ((), ())),
        preferred_element_type=jnp.float32)
    c_ref[0] = c + b_ref[...]


def _make_sc_gather_max(bn, n, fout, k, pts_w, chunk):
    nsteps = pts_w // chunk
    mesh = plsc.VectorSubcoreMesh(core_axis_name="c", subcore_axis_name="s")

    @functools.partial(
        pl.kernel, mesh=mesh,
        out_type=jax.ShapeDtypeStruct((bn, fout), jnp.float32),
        compiler_params=pltpu.CompilerParams(use_tc_tiling_on_sc=False),
        scratch_types=[
            pltpu.VMEM((k, chunk), jnp.int32),
            pltpu.VMEM((k * chunk, fout), jnp.float32),
            pltpu.VMEM((chunk, fout), jnp.float32),
            pltpu.VMEM((chunk, fout), jnp.float32),
            pltpu.SemaphoreType.DMA,
        ],
    )
    def sc_fn(y_hbm, idxj_hbm, c_hbm, out_hbm, idx_v, rows_v, c_v, out_v, sem):
        wid = lax.axis_index("s") * 2 + lax.axis_index("c")
        base = wid * pts_w
        b = base // n

        def chunk_body(g, carry):
            pt = base + g * chunk
            n0 = pt - b * n
            pltpu.sync_copy(
                idxj_hbm.at[pl.ds(b * _KPAD, k), pl.ds(n0, chunk)], idx_v)
            copies = [
                pltpu.async_copy(
                    y_hbm.at[idx_v.at[j]],
                    rows_v.at[pl.ds(j * chunk, chunk)], sem)
                for j in range(k)
            ]
            pltpu.sync_copy(c_hbm.at[pl.ds(pt, chunk)], c_v)
            for cp in copies:
                cp.wait()

            def point_body(p, carry2):
                for l in range(fout // 16):
                    sl = pl.ds(l * 16, 16)
                    acc = rows_v[p, sl]
                    for j in range(1, k):
                        acc = jnp.maximum(acc, rows_v[j * chunk + p, sl])
                    out_v[p, sl] = acc + c_v[p, sl]
                return carry2

            lax.fori_loop(0, chunk, point_body, 0)
            pltpu.sync_copy(out_v, out_hbm.at[pl.ds(pt, chunk)])
            return carry

        lax.fori_loop(0, nsteps, chunk_body, 0)

    return sc_fn


def _half(xt, w2t, wdt, b2, sc_fn, tile):
    B, N, Fin = xt.shape
    Fout = w2t.shape[1]
    idx, y, c = pl.pallas_call(
        functools.partial(_knn_tc_kernel, tile=tile, k=_K),
        grid=(B, N // tile),
        in_specs=[
            pl.BlockSpec((1, N, Fin), lambda bb, ii: (bb, 0, 0)),
            pl.BlockSpec((Fin, Fout), lambda bb, ii: (0, 0)),
            pl.BlockSpec((Fin, Fout), lambda bb, ii: (0, 0)),
            pl.BlockSpec((1, Fout), lambda bb, ii: (0, 0)),
        ],
        out_specs=[
            pl.BlockSpec((1, tile, _KPAD), lambda bb, ii: (bb, ii, 0)),
            pl.BlockSpec((1, tile, Fout), lambda bb, ii: (bb, ii, 0)),
            pl.BlockSpec((1, tile, Fout), lambda bb, ii: (bb, ii, 0)),
        ],
        out_shape=[
            jax.ShapeDtypeStruct((B, N, _KPAD), jnp.int32),
            jax.ShapeDtypeStruct((B, N, Fout), jnp.float32),
            jax.ShapeDtypeStruct((B, N, Fout), jnp.float32),
        ],
    )(xt, w2t, wdt, b2)

    bn = B * N
    idxj = jnp.transpose(idx, (0, 2, 1)).reshape(B * _KPAD, N)
    return sc_fn(y.reshape(bn, Fout), idxj, c.reshape(bn, Fout))


def kernel(x, W, b):
    B, Fin, N = x.shape
    Fout = W.shape[0]
    tile = 512 if N % 512 == 0 else N

    xt = jnp.transpose(x, (0, 2, 1))
    W1, W2 = W[:, :Fin], W[:, Fin:]
    w2t = jnp.transpose(W2)
    wdt = jnp.transpose(W1 - W2)
    b2 = b[None, :]

    bh = B // 4 if B % 4 == 0 else B
    nh = B // bh
    bn = bh * N
    sc_fn = _make_sc_gather_max(bn, N, Fout, _K, bn // 32, 32)
    outs = [_half(xt[h * bh:(h + 1) * bh], w2t, wdt, b2, sc_fn, tile)
            for h in range(nh)]
    out = jnp.concatenate(outs, axis=0)

    return jnp.transpose(out.reshape(B, N, Fout), (0, 2, 1))


# speedup vs baseline: 1.1001x; 1.0084x over previous
import functools

import jax
import jax.numpy as jnp
from jax import lax
from jax.experimental import pallas as pl
from jax.experimental.pallas import tpu as pltpu
from jax.experimental.pallas import tpu_sc as plsc

_K = 20
_KPAD = 32


def _knn_tc_kernel(xt_ref, w2t_ref, wdt_ref, b_ref, idx_ref, y_ref, c_ref,
                   *, tile, k):
    n = xt_ref.shape[1]
    i = pl.program_id(1)
    bb = pl.program_id(0)

    xt = xt_ref[0]
    xt_tile = xt_ref[0, pl.ds(i * tile, tile), :]

    sq = jnp.sum(xt * xt, axis=1)
    sq_tile = jnp.sum(xt_tile * xt_tile, axis=1)
    xi = -2.0 * lax.dot_general(
        xt_tile, xt, (((1,), (1,)), ((), ())),
        preferred_element_type=jnp.float32)
    dist = (xi + sq_tile[:, None]) + sq[None, :]

    colf = lax.broadcasted_iota(jnp.int32, (tile, n), 1).astype(jnp.float32)
    col32 = lax.broadcasted_iota(jnp.int32, (tile, _KPAD), 1)

    def extract_min(d):
        m = jnp.min(d, axis=1, keepdims=True)
        idxf = jnp.min(jnp.where(d == m, colf, float(n)), axis=1,
                       keepdims=True)
        return idxf, jnp.where(colf == idxf, jnp.inf, d)

    _, dist = extract_min(dist)

    def body(j, carry):
        d, acc = carry
        idxf, d = extract_min(d)
        idx = idxf.astype(jnp.int32)
        acc = jnp.where(col32 == j, idx + bb * n, acc)
        return d, acc

    acc0 = jnp.zeros((tile, _KPAD), dtype=jnp.int32)
    _, idxacc = lax.fori_loop(0, k, body, (dist, acc0))
    idx_ref[0] = idxacc

    y_ref[0] = lax.dot_general(
        xt_tile, w2t_ref[...], (((1,), (0,)), ((), ())),
        preferred_element_type=jnp.float32)
    c = lax.dot_general(
        xt_tile, wdt_ref[...], (((1,), (0,)), ((), ())),
        preferred_element_type=jnp.float32)
    c_ref[0] = c + b_ref[...]


def _make_sc_gather_max(bn, n, fout, k, pts_w, chunk):
    nsteps = pts_w // chunk
    mesh = plsc.VectorSubcoreMesh(core_axis_name="c", subcore_axis_name="s")

    @functools.partial(
        pl.kernel, mesh=mesh,
        out_type=jax.ShapeDtypeStruct((bn, fout), jnp.float32),
        compiler_params=pltpu.CompilerParams(use_tc_tiling_on_sc=False),
        scratch_types=[
            pltpu.VMEM((k, chunk), jnp.int32),
            pltpu.VMEM((k * chunk, fout), jnp.float32),
            pltpu.VMEM((chunk, fout), jnp.float32),
            pltpu.VMEM((chunk, fout), jnp.float32),
            pltpu.SemaphoreType.DMA,
        ],
    )
    def sc_fn(y_hbm, idxj_hbm, c_hbm, out_hbm, idx_v, rows_v, c_v, out_v, sem):
        wid = lax.axis_index("s") * 2 + lax.axis_index("c")
        base = wid * pts_w
        b = base // n

        def chunk_body(g, carry):
            pt = base + g * chunk
            n0 = pt - b * n
            pltpu.sync_copy(
                idxj_hbm.at[pl.ds(b * _KPAD, k), pl.ds(n0, chunk)], idx_v)
            copies = [
                pltpu.async_copy(
                    y_hbm.at[idx_v.at[j]],
                    rows_v.at[pl.ds(j * chunk, chunk)], sem)
                for j in range(k)
            ]
            pltpu.sync_copy(c_hbm.at[pl.ds(pt, chunk)], c_v)
            for cp in copies:
                cp.wait()

            def point_body(p, carry2):
                for l in range(fout // 16):
                    sl = pl.ds(l * 16, 16)
                    acc = rows_v[p, sl]
                    for j in range(1, k):
                        acc = jnp.maximum(acc, rows_v[j * chunk + p, sl])
                    out_v[p, sl] = acc + c_v[p, sl]
                return carry2

            lax.fori_loop(0, chunk, point_body, 0)
            pltpu.sync_copy(out_v, out_hbm.at[pl.ds(pt, chunk)])
            return carry

        lax.fori_loop(0, nsteps, chunk_body, 0)

    return sc_fn


def _half(xt, w2t, wdt, b2, sc_fn, tile):
    B, N, Fin = xt.shape
    Fout = w2t.shape[1]
    idx, y, c = pl.pallas_call(
        functools.partial(_knn_tc_kernel, tile=tile, k=_K),
        grid=(B, N // tile),
        in_specs=[
            pl.BlockSpec((1, N, Fin), lambda bb, ii: (bb, 0, 0)),
            pl.BlockSpec((Fin, Fout), lambda bb, ii: (0, 0)),
            pl.BlockSpec((Fin, Fout), lambda bb, ii: (0, 0)),
            pl.BlockSpec((1, Fout), lambda bb, ii: (0, 0)),
        ],
        out_specs=[
            pl.BlockSpec((1, tile, _KPAD), lambda bb, ii: (bb, ii, 0)),
            pl.BlockSpec((1, tile, Fout), lambda bb, ii: (bb, ii, 0)),
            pl.BlockSpec((1, tile, Fout), lambda bb, ii: (bb, ii, 0)),
        ],
        out_shape=[
            jax.ShapeDtypeStruct((B, N, _KPAD), jnp.int32),
            jax.ShapeDtypeStruct((B, N, Fout), jnp.float32),
            jax.ShapeDtypeStruct((B, N, Fout), jnp.float32),
        ],
    )(xt, w2t, wdt, b2)

    bn = B * N
    idxj = jnp.transpose(idx, (0, 2, 1)).reshape(B * _KPAD, N)
    return sc_fn(y.reshape(bn, Fout), idxj, c.reshape(bn, Fout))


def kernel(x, W, b):
    B, Fin, N = x.shape
    Fout = W.shape[0]
    tile = 512 if N % 512 == 0 else N

    xt = jnp.transpose(x, (0, 2, 1))
    W1, W2 = W[:, :Fin], W[:, Fin:]
    w2t = jnp.transpose(W2)
    wdt = jnp.transpose(W1 - W2)
    b2 = b[None, :]

    bh = 1
    nh = B // bh
    bn = bh * N
    sc_fn = _make_sc_gather_max(bn, N, Fout, _K, bn // 32, 32)
    outs = [_half(xt[h * bh:(h + 1) * bh], w2t, wdt, b2, sc_fn, tile)
            for h in range(nh)]
    out = jnp.concatenate(outs, axis=0)

    return jnp.transpose(out.reshape(B, N, Fout), (0, 2, 1))
